# baseline (speedup 1.0000x reference)
import jax
import jax.numpy as jnp
from jax.experimental import pallas as pl

W = 256
H = 16
M = 144
MM = M * M
ROWS = 529
GCHUNK = 2304
NB = 4


def _gather_body(idx_ref, table_ref, out_ref):
    idx = idx_ref[0]
    iota = jax.lax.broadcasted_iota(jnp.int32, (1, ROWS), 1)
    onehot = (idx == iota).astype(jnp.float32)
    out_ref[0] = jnp.dot(onehot, table_ref[...],
                         preferred_element_type=jnp.float32)


def _add_body(att_ref, bias_ref, out_ref):
    out_ref[...] = att_ref[...] + jnp.tile(bias_ref[...], (NB, 1))


def kernel(att_scores, bias_table, relative_position_index):
    idx = relative_position_index.reshape(MM // GCHUNK, GCHUNK, 1)
    gathered = pl.pallas_call(
        _gather_body,
        grid=(MM // GCHUNK,),
        in_specs=[
            pl.BlockSpec((1, GCHUNK, 1), lambda c: (c, 0, 0)),
            pl.BlockSpec((ROWS, H), lambda c: (0, 0)),
        ],
        out_specs=pl.BlockSpec((1, GCHUNK, H), lambda c: (c, 0, 0)),
        out_shape=jax.ShapeDtypeStruct((MM // GCHUNK, GCHUNK, H), jnp.float32),
    )(idx, bias_table)

    bias = gathered.reshape(MM, H).T

    att2 = att_scores.reshape(W * H, MM)
    out2 = pl.pallas_call(
        _add_body,
        grid=(W // NB,),
        in_specs=[
            pl.BlockSpec((NB * H, MM), lambda w: (w, 0)),
            pl.BlockSpec((H, MM), lambda w: (0, 0)),
        ],
        out_specs=pl.BlockSpec((NB * H, MM), lambda w: (w, 0)),
        out_shape=jax.ShapeDtypeStruct((W * H, MM), jnp.float32),
    )(att2, bias)
    return out2.reshape(W, H, M, M)


# speedup vs baseline: 2.2300x; 2.2300x over previous
import jax
import jax.numpy as jnp
from jax.experimental import pallas as pl

W = 256
H = 16
M = 144
ROWS = 529
IB = 8
NB = 4


def _gather_body(rpi_ref, btT_ref, out_ref):
    iota = jax.lax.broadcasted_iota(jnp.int32, (ROWS, M), 0)
    btT = btT_ref[...]
    for rr in range(IB):
        onehot = (rpi_ref[rr:rr + 1, :] == iota).astype(jnp.float32)
        out_ref[:, rr, :] = jnp.dot(btT, onehot,
                                    preferred_element_type=jnp.float32)


def _add_body(att_ref, bias_ref, out_ref):
    out_ref[...] = att_ref[...] + bias_ref[...][None]


def kernel(att_scores, bias_table, relative_position_index):
    bias = pl.pallas_call(
        _gather_body,
        grid=(M // IB,),
        in_specs=[
            pl.BlockSpec((IB, M), lambda c: (c, 0)),
            pl.BlockSpec((H, ROWS), lambda c: (0, 0)),
        ],
        out_specs=pl.BlockSpec((H, IB, M), lambda c: (0, c, 0)),
        out_shape=jax.ShapeDtypeStruct((H, M, M), jnp.float32),
    )(relative_position_index, bias_table.T)

    return pl.pallas_call(
        _add_body,
        grid=(W // NB,),
        in_specs=[
            pl.BlockSpec((NB, H, M, M), lambda w: (w, 0, 0, 0)),
            pl.BlockSpec((H, M, M), lambda w: (0, 0, 0)),
        ],
        out_specs=pl.BlockSpec((NB, H, M, M), lambda w: (w, 0, 0, 0)),
        out_shape=jax.ShapeDtypeStruct((W, H, M, M), jnp.float32),
    )(att_scores, bias)
